# baseline (speedup 1.0000x reference)
import functools

import jax
import jax.numpy as jnp
from jax import lax
from jax.experimental import pallas as pl
from jax.experimental.pallas import tpu as pltpu
from jax.experimental.pallas import tpu_sc as plsc

KCB = 8192
DDIM = 256
BTOK = 8192
BETA_W = 0.25

BM = 256

NC = 2
NS = 16
LANES = 16
NW = NC * NS
BPW = BTOK // NW
SUB = 128
NSUB = BPW // SUB


def _distance_argmin_body(z_ref, c_ref, idx_ref):
    z = z_ref[...]
    c = c_ref[...]
    p = lax.dot_general(z, c, (((1,), (1,)), ((), ())),
                        preferred_element_type=jnp.float32)
    z_sq = jnp.sum(z * z, axis=1, keepdims=True)
    c_sq = lax.dot_general(jnp.ones((1, DDIM), jnp.float32), c * c,
                           (((1,), (1,)), ((), ())),
                           precision=lax.Precision.HIGHEST,
                           preferred_element_type=jnp.float32)
    dist = z_sq - 2.0 * p + c_sq
    m = jnp.min(dist, axis=1, keepdims=True)
    ks = lax.broadcasted_iota(jnp.int32, (BM, KCB), 1)
    idx = jnp.min(jnp.where(dist == m, ks, KCB), axis=1, keepdims=True)
    idx_ref[...] = idx


def _distance_argmin_call(z_e, codebook):
    nb = BTOK // BM
    return pl.pallas_call(
        _distance_argmin_body,
        grid=(nb,),
        in_specs=[
            pl.BlockSpec((BM, DDIM), lambda i: (i, 0)),
            pl.BlockSpec((KCB, DDIM), lambda i: (0, 0)),
        ],
        out_specs=pl.BlockSpec((BM, 1), lambda i: (i, 0)),
        out_shape=jax.ShapeDtypeStruct((BTOK, 1), jnp.int32),
        compiler_params=pltpu.CompilerParams(
            dimension_semantics=("arbitrary",)),
    )(z_e, codebook)


@functools.lru_cache(maxsize=1)
def _sc_gather_loss_kernel():
    @functools.partial(
        pl.kernel,
        out_type=(
            jax.ShapeDtypeStruct((BTOK, DDIM), jnp.float32),
            jax.ShapeDtypeStruct((NW, LANES), jnp.float32),
        ),
        mesh=plsc.VectorSubcoreMesh(
            core_axis_name="c", subcore_axis_name="s",
            num_cores=NC, num_subcores=NS),
        scratch_types=[
            pltpu.VMEM((BPW,), jnp.int32),
            pltpu.VMEM((SUB, DDIM), jnp.float32),
            pltpu.VMEM((SUB, DDIM), jnp.float32),
            pltpu.VMEM((LANES,), jnp.float32),
            pltpu.SemaphoreType.DMA,
        ],
    )
    def _sc_gather_loss(cb_hbm, idx_hbm, ze_hbm, zq_hbm, part_hbm,
                        idx_v, rows_v, ze_v, acc_v, sem):
        wid = lax.axis_index("s") * NC + lax.axis_index("c")
        base = wid * BPW
        pltpu.sync_copy(idx_hbm.at[pl.ds(base, BPW)], idx_v)
        acc = jnp.zeros((LANES,), jnp.float32)
        for sub in range(NSUB):
            pltpu.async_copy(
                cb_hbm.at[idx_v.at[pl.ds(sub * SUB, SUB)]], rows_v, sem).wait()
            pltpu.sync_copy(ze_hbm.at[pl.ds(base + sub * SUB, SUB)], ze_v)

            def body(i, a):
                r = i // (DDIM // LANES)
                col = (i % (DDIM // LANES)) * LANES
                zq = rows_v[r, pl.ds(col, LANES)]
                ze = ze_v[r, pl.ds(col, LANES)]
                d = zq - ze
                rows_v[r, pl.ds(col, LANES)] = ze + d
                return a + d * d

            acc = lax.fori_loop(0, SUB * (DDIM // LANES), body, acc)
            pltpu.sync_copy(rows_v, zq_hbm.at[pl.ds(base + sub * SUB, SUB)])
        acc_v[...] = acc
        pltpu.sync_copy(acc_v, part_hbm.at[wid])

    return _sc_gather_loss


def kernel(z_e, codebook):
    idx2 = _distance_argmin_call(z_e, codebook)
    indices = idx2.reshape(BTOK)
    z_q_st, parts = _sc_gather_loss_kernel()(codebook, indices, z_e)
    m = jnp.sum(parts) / (BTOK * DDIM)
    vq_loss = m + BETA_W * m
    return (z_q_st, indices, vq_loss)


# speedup vs baseline: 2.1514x; 2.1514x over previous
import functools

import jax
import jax.numpy as jnp
from jax import lax
from jax.experimental import pallas as pl
from jax.experimental.pallas import tpu as pltpu
from jax.experimental.pallas import tpu_sc as plsc

KCB = 8192
DDIM = 256
BTOK = 8192
BETA_W = 0.25

BM = 256

NC = 2
NS = 16
LANES = 16
NW = NC * NS
BPW = BTOK // NW
SUB = 128
NSUB = BPW // SUB


def _csq_body(c_ref, csq_ref):
    c = c_ref[...]
    csq_ref[...] = lax.dot_general(jnp.ones((8, DDIM), jnp.float32), c * c,
                                   (((1,), (1,)), ((), ())),
                                   precision=lax.Precision.HIGHEST,
                                   preferred_element_type=jnp.float32)


def _csq_call(codebook):
    return pl.pallas_call(
        _csq_body,
        out_shape=jax.ShapeDtypeStruct((8, KCB), jnp.float32),
    )(codebook)


def _distance_argmin_body(zb_ref, cb_ref, z_ref, csq_ref, idx_ref):
    p = lax.dot_general(zb_ref[...], cb_ref[...], (((1,), (1,)), ((), ())),
                        preferred_element_type=jnp.float32)
    z = z_ref[...]
    z_sq = jnp.sum(z * z, axis=1, keepdims=True)
    dist = (z_sq - 2.0 * p) + csq_ref[0:1, :]
    m = jnp.min(dist, axis=1, keepdims=True)
    ks = lax.broadcasted_iota(jnp.int32, (BM, KCB), 1)
    idx = jnp.min(jnp.where(dist == m, ks, KCB), axis=1, keepdims=True)
    idx_ref[...] = idx


def _distance_argmin_call(z_e, codebook):
    nb = BTOK // BM
    csq = _csq_call(codebook)
    z_bf = z_e.astype(jnp.bfloat16)
    c_bf = codebook.astype(jnp.bfloat16)
    return pl.pallas_call(
        _distance_argmin_body,
        grid=(nb,),
        in_specs=[
            pl.BlockSpec((BM, DDIM), lambda i: (i, 0)),
            pl.BlockSpec((KCB, DDIM), lambda i: (0, 0)),
            pl.BlockSpec((BM, DDIM), lambda i: (i, 0)),
            pl.BlockSpec((8, KCB), lambda i: (0, 0)),
        ],
        out_specs=pl.BlockSpec((BM, 1), lambda i: (i, 0)),
        out_shape=jax.ShapeDtypeStruct((BTOK, 1), jnp.int32),
        compiler_params=pltpu.CompilerParams(
            dimension_semantics=("arbitrary",)),
    )(z_bf, c_bf, z_e, csq)


@functools.lru_cache(maxsize=1)
def _sc_gather_loss_kernel():
    @functools.partial(
        pl.kernel,
        out_type=(
            jax.ShapeDtypeStruct((BTOK, DDIM), jnp.float32),
            jax.ShapeDtypeStruct((NW, LANES), jnp.float32),
        ),
        mesh=plsc.VectorSubcoreMesh(
            core_axis_name="c", subcore_axis_name="s",
            num_cores=NC, num_subcores=NS),
        scratch_types=[
            pltpu.VMEM((BPW,), jnp.int32),
            pltpu.VMEM((SUB, DDIM), jnp.float32),
            pltpu.VMEM((SUB, DDIM), jnp.float32),
            pltpu.VMEM((LANES,), jnp.float32),
            pltpu.SemaphoreType.DMA,
        ],
    )
    def _sc_gather_loss(cb_hbm, idx_hbm, ze_hbm, zq_hbm, part_hbm,
                        idx_v, rows_v, ze_v, acc_v, sem):
        wid = lax.axis_index("s") * NC + lax.axis_index("c")
        base = wid * BPW
        pltpu.sync_copy(idx_hbm.at[pl.ds(base, BPW)], idx_v)
        acc = jnp.zeros((LANES,), jnp.float32)
        for sub in range(NSUB):
            pltpu.async_copy(
                cb_hbm.at[idx_v.at[pl.ds(sub * SUB, SUB)]], rows_v, sem).wait()
            pltpu.sync_copy(ze_hbm.at[pl.ds(base + sub * SUB, SUB)], ze_v)

            def body(i, a):
                r = i // (DDIM // LANES)
                col = (i % (DDIM // LANES)) * LANES
                zq = rows_v[r, pl.ds(col, LANES)]
                ze = ze_v[r, pl.ds(col, LANES)]
                d = zq - ze
                rows_v[r, pl.ds(col, LANES)] = ze + d
                return a + d * d

            acc = lax.fori_loop(0, SUB * (DDIM // LANES), body, acc)
            pltpu.sync_copy(rows_v, zq_hbm.at[pl.ds(base + sub * SUB, SUB)])
        acc_v[...] = acc
        pltpu.sync_copy(acc_v, part_hbm.at[wid])

    return _sc_gather_loss


def kernel(z_e, codebook):
    idx2 = _distance_argmin_call(z_e, codebook)
    indices = idx2.reshape(BTOK)
    z_q_st, parts = _sc_gather_loss_kernel()(codebook, indices, z_e)
    m = jnp.sum(parts) / (BTOK * DDIM)
    vq_loss = m + BETA_W * m
    return (z_q_st, indices, vq_loss)


# speedup vs baseline: 2.6085x; 1.2125x over previous
import functools

import jax
import jax.numpy as jnp
from jax import lax
from jax.experimental import pallas as pl
from jax.experimental.pallas import tpu as pltpu
from jax.experimental.pallas import tpu_sc as plsc

KCB = 8192
DDIM = 256
BTOK = 8192
BETA_W = 0.25

BM = 256

NC = 2
NS = 16
LANES = 16
NW = NC * NS
BPW = BTOK // NW
SUB = 128
NSUB = BPW // SUB


def _csq_body(c_ref, csq_ref):
    c = c_ref[...]
    csq_ref[...] = lax.dot_general(jnp.ones((8, DDIM), jnp.float32), c * c,
                                   (((1,), (1,)), ((), ())),
                                   precision=lax.Precision.HIGHEST,
                                   preferred_element_type=jnp.float32)


def _csq_call(codebook):
    return pl.pallas_call(
        _csq_body,
        out_shape=jax.ShapeDtypeStruct((8, KCB), jnp.float32),
    )(codebook)


GW = 128
NG = KCB // GW


def _distance_argmin_body(cb_ref, z_ref, csq_ref, ks_ref, idx_ref):
    z = z_ref[...]
    zb = z.astype(jnp.bfloat16)
    p2 = lax.dot_general(zb, cb_ref[...], (((1,), (1,)), ((), ())),
                         preferred_element_type=jnp.float32)
    z_sq = jnp.sum(z * z, axis=1, keepdims=True)
    run_m = jnp.full((BM, GW), jnp.inf, jnp.float32)
    run_g = jnp.zeros((BM, GW), jnp.float32)
    for g in range(NG):
        d = (z_sq + p2[:, g * GW:(g + 1) * GW]) + csq_ref[0:1, g * GW:(g + 1) * GW]
        lt = d < run_m
        run_m = jnp.minimum(run_m, d)
        run_g = jnp.where(lt, jnp.float32(g), run_g)
    kcand = run_g * jnp.float32(GW) + ks_ref[0:1, 0:GW]
    gm = jnp.min(run_m, axis=1, keepdims=True)
    idxf = jnp.min(jnp.where(run_m == gm, kcand, jnp.float32(KCB)),
                   axis=1, keepdims=True)
    idx_ref[...] = idxf.astype(jnp.int32)


def _distance_argmin_call(z_e, codebook):
    nb = BTOK // BM
    csq = _csq_call(codebook)
    m2c_bf = (-2.0 * codebook).astype(jnp.bfloat16)
    ks_row = jnp.broadcast_to(
        jnp.arange(KCB, dtype=jnp.float32)[None, :], (8, KCB))
    return pl.pallas_call(
        _distance_argmin_body,
        grid=(nb,),
        in_specs=[
            pl.BlockSpec((KCB, DDIM), lambda i: (0, 0)),
            pl.BlockSpec((BM, DDIM), lambda i: (i, 0)),
            pl.BlockSpec((8, KCB), lambda i: (0, 0)),
            pl.BlockSpec((8, KCB), lambda i: (0, 0)),
        ],
        out_specs=pl.BlockSpec((BM, 1), lambda i: (i, 0)),
        out_shape=jax.ShapeDtypeStruct((BTOK, 1), jnp.int32),
        compiler_params=pltpu.CompilerParams(
            dimension_semantics=("arbitrary",)),
    )(m2c_bf, z_e, csq, ks_row)


@functools.lru_cache(maxsize=1)
def _sc_gather_loss_kernel():
    @functools.partial(
        pl.kernel,
        out_type=(
            jax.ShapeDtypeStruct((BTOK, DDIM), jnp.float32),
            jax.ShapeDtypeStruct((NW, LANES), jnp.float32),
        ),
        mesh=plsc.VectorSubcoreMesh(
            core_axis_name="c", subcore_axis_name="s",
            num_cores=NC, num_subcores=NS),
        scratch_types=[
            pltpu.VMEM((BPW,), jnp.int32),
            pltpu.VMEM((SUB, DDIM), jnp.float32),
            pltpu.VMEM((SUB, DDIM), jnp.float32),
            pltpu.VMEM((LANES,), jnp.float32),
            pltpu.SemaphoreType.DMA,
        ],
    )
    def _sc_gather_loss(cb_hbm, idx_hbm, ze_hbm, zq_hbm, part_hbm,
                        idx_v, rows_v, ze_v, acc_v, sem):
        wid = lax.axis_index("s") * NC + lax.axis_index("c")
        base = wid * BPW
        pltpu.sync_copy(idx_hbm.at[pl.ds(base, BPW)], idx_v)
        acc = jnp.zeros((LANES,), jnp.float32)
        for sub in range(NSUB):
            pltpu.async_copy(
                cb_hbm.at[idx_v.at[pl.ds(sub * SUB, SUB)]], rows_v, sem).wait()
            pltpu.sync_copy(ze_hbm.at[pl.ds(base + sub * SUB, SUB)], ze_v)

            @plsc.parallel_loop(0, SUB * (DDIM // LANES), unroll=8, carry=acc)
            def acc(i, a):
                r = i // (DDIM // LANES)
                col = (i % (DDIM // LANES)) * LANES
                zq = rows_v[r, pl.ds(col, LANES)]
                ze = ze_v[r, pl.ds(col, LANES)]
                d = zq - ze
                rows_v[r, pl.ds(col, LANES)] = ze + d
                return a + d * d
            pltpu.sync_copy(rows_v, zq_hbm.at[pl.ds(base + sub * SUB, SUB)])
        acc_v[...] = acc
        pltpu.sync_copy(acc_v, part_hbm.at[wid])

    return _sc_gather_loss


def kernel(z_e, codebook):
    idx2 = _distance_argmin_call(z_e, codebook)
    indices = idx2.reshape(BTOK)
    z_q_st, parts = _sc_gather_loss_kernel()(codebook, indices, z_e)
    m = jnp.sum(parts) / (BTOK * DDIM)
    vq_loss = m + BETA_W * m
    return (z_q_st, indices, vq_loss)


# speedup vs baseline: 3.4337x; 1.3163x over previous
import functools

import jax
import jax.numpy as jnp
from jax import lax
from jax.experimental import pallas as pl
from jax.experimental.pallas import tpu as pltpu
from jax.experimental.pallas import tpu_sc as plsc

KCB = 8192
DDIM = 256
BTOK = 8192
BETA_W = 0.25

BM = 512
GW = 128
NG = KCB // GW

NC = 2
NS = 16
LANES = 16
NW = NC * NS
BPW = BTOK // NW
SUB = 64
NSUB = BPW // SUB


def _distance_argmin_body(c_ref, z_ref, idx_ref, m2c_s, csq_s):
    i = pl.program_id(0)

    @pl.when(i == 0)
    def _prep():
        c = c_ref[...]
        m2c_s[...] = (-2.0 * c).astype(jnp.bfloat16)
        csq_s[...] = lax.dot_general(
            jnp.ones((8, DDIM), jnp.float32), c * c,
            (((1,), (1,)), ((), ())),
            precision=lax.Precision.HIGHEST,
            preferred_element_type=jnp.float32)

    z = z_ref[...]
    zb = z.astype(jnp.bfloat16)
    p2 = lax.dot_general(zb, m2c_s[...], (((1,), (1,)), ((), ())),
                         preferred_element_type=jnp.float32)
    z_sq = jnp.sum(z * z, axis=1, keepdims=True)
    run_m = jnp.full((BM, GW), jnp.inf, jnp.float32)
    run_g = jnp.zeros((BM, GW), jnp.float32)
    for g in range(NG):
        d = (z_sq + p2[:, g * GW:(g + 1) * GW]) + csq_s[0:1, g * GW:(g + 1) * GW]
        lt = d < run_m
        run_m = jnp.minimum(run_m, d)
        run_g = jnp.where(lt, jnp.float32(g), run_g)
    lane = lax.broadcasted_iota(jnp.int32, (BM, GW), 1).astype(jnp.float32)
    kcand = run_g * jnp.float32(GW) + lane
    gm = jnp.min(run_m, axis=1, keepdims=True)
    idxf = jnp.min(jnp.where(run_m == gm, kcand, jnp.float32(KCB)),
                   axis=1, keepdims=True)
    idx_ref[...] = idxf.astype(jnp.int32)


def _distance_argmin_call(z_e, codebook):
    nb = BTOK // BM
    return pl.pallas_call(
        _distance_argmin_body,
        grid=(nb,),
        in_specs=[
            pl.BlockSpec((KCB, DDIM), lambda i: (0, 0)),
            pl.BlockSpec((BM, DDIM), lambda i: (i, 0)),
        ],
        out_specs=pl.BlockSpec((BM, 1), lambda i: (i, 0)),
        out_shape=jax.ShapeDtypeStruct((BTOK, 1), jnp.int32),
        scratch_shapes=[
            pltpu.VMEM((KCB, DDIM), jnp.bfloat16),
            pltpu.VMEM((8, KCB), jnp.float32),
        ],
        compiler_params=pltpu.CompilerParams(
            dimension_semantics=("arbitrary",)),
    )(codebook, z_e)


@functools.lru_cache(maxsize=1)
def _sc_gather_loss_kernel():
    @functools.partial(
        pl.kernel,
        out_type=(
            jax.ShapeDtypeStruct((BTOK, DDIM), jnp.float32),
            jax.ShapeDtypeStruct((NW, LANES), jnp.float32),
        ),
        mesh=plsc.VectorSubcoreMesh(
            core_axis_name="c", subcore_axis_name="s",
            num_cores=NC, num_subcores=NS),
        scratch_types=[
            pltpu.VMEM((BPW,), jnp.int32),
            pltpu.VMEM((SUB, DDIM), jnp.float32),
            pltpu.VMEM((SUB, DDIM), jnp.float32),
            pltpu.VMEM((SUB, DDIM), jnp.float32),
            pltpu.VMEM((SUB, DDIM), jnp.float32),
            pltpu.VMEM((SUB, DDIM), jnp.float32),
            pltpu.VMEM((SUB, DDIM), jnp.float32),
            pltpu.VMEM((LANES,), jnp.float32),
            pltpu.SemaphoreType.DMA,
            pltpu.SemaphoreType.DMA,
            pltpu.SemaphoreType.DMA,
            pltpu.SemaphoreType.DMA,
            pltpu.SemaphoreType.DMA,
            pltpu.SemaphoreType.DMA,
        ],
    )
    def _sc_gather_loss(cb_hbm, idx_hbm, ze_hbm, zq_hbm, part_hbm,
                        idx_v, rows0, rows1, ze0, ze1, st0, st1, acc_v,
                        g0, g1, zs0, zs1, os0, os1):
        wid = lax.axis_index("s") * NC + lax.axis_index("c")
        base = wid * BPW
        pltpu.sync_copy(idx_hbm.at[pl.ds(base, BPW)], idx_v)
        rows = (rows0, rows1)
        zeb = (ze0, ze1)
        stb = (st0, st1)
        gsem = (g0, g1)
        zsem = (zs0, zs1)
        osem = (os0, os1)
        gh = [None] * NSUB
        zh = [None] * NSUB
        oh = [None] * NSUB

        def issue(s):
            b = s % 2
            gh[s] = pltpu.async_copy(
                cb_hbm.at[idx_v.at[pl.ds(s * SUB, SUB)]], rows[b], gsem[b])
            zh[s] = pltpu.async_copy(
                ze_hbm.at[pl.ds(base + s * SUB, SUB)], zeb[b], zsem[b])

        issue(0)
        accs = (jnp.zeros((LANES,), jnp.float32),
                jnp.zeros((LANES,), jnp.float32))
        for s in range(NSUB):
            b = s % 2
            gh[s].wait()
            zh[s].wait()
            if s + 1 < NSUB:
                issue(s + 1)
            if s >= 2:
                oh[s - 2].wait()

            @plsc.parallel_loop(0, SUB, unroll=2, carry=accs)
            def accs(r, a):
                a0, a1 = a
                for cc in range(DDIM // LANES):
                    zq = rows[b][r, pl.ds(cc * LANES, LANES)]
                    ze = zeb[b][r, pl.ds(cc * LANES, LANES)]
                    d = zq - ze
                    stb[b][r, pl.ds(cc * LANES, LANES)] = ze + d
                    if cc % 2 == 0:
                        a0 = a0 + d * d
                    else:
                        a1 = a1 + d * d
                return (a0, a1)

            oh[s] = pltpu.async_copy(
                stb[b], zq_hbm.at[pl.ds(base + s * SUB, SUB)], osem[b])
        oh[NSUB - 2].wait()
        oh[NSUB - 1].wait()
        acc_v[...] = accs[0] + accs[1]
        pltpu.sync_copy(acc_v, part_hbm.at[wid])

    return _sc_gather_loss


def kernel(z_e, codebook):
    idx2 = _distance_argmin_call(z_e, codebook)
    indices = idx2.reshape(BTOK)
    z_q_st, parts = _sc_gather_loss_kernel()(codebook, indices, z_e)
    m = jnp.sum(parts) / (BTOK * DDIM)
    vq_loss = m + BETA_W * m
    return (z_q_st, indices, vq_loss)


# speedup vs baseline: 3.5253x; 1.0267x over previous
import functools

import jax
import jax.numpy as jnp
from jax import lax
from jax.experimental import pallas as pl
from jax.experimental.pallas import tpu as pltpu
from jax.experimental.pallas import tpu_sc as plsc

KCB = 8192
DDIM = 256
BTOK = 8192
BETA_W = 0.25

BM = 1024
GW = 128
NG = KCB // GW

NC = 2
NS = 16
LANES = 16
NW = NC * NS
BPW = BTOK // NW
SUB = 64
NSUB = BPW // SUB


def _distance_argmin_body(c_ref, z_ref, idx_ref, m2c_s, csq_s):
    i = pl.program_id(0)

    @pl.when(i == 0)
    def _prep():
        c = c_ref[...]
        m2c_s[...] = (-2.0 * c).astype(jnp.bfloat16)
        csq_s[...] = lax.dot_general(
            jnp.ones((8, DDIM), jnp.float32), c * c,
            (((1,), (1,)), ((), ())),
            precision=lax.Precision.HIGHEST,
            preferred_element_type=jnp.float32)

    z = z_ref[...]
    zb = z.astype(jnp.bfloat16)
    p2 = lax.dot_general(zb, m2c_s[...], (((1,), (1,)), ((), ())),
                         preferred_element_type=jnp.float32)
    z_sq = jnp.sum(z * z, axis=1, keepdims=True)
    run_m = jnp.full((BM, GW), jnp.inf, jnp.float32)
    run_g = jnp.zeros((BM, GW), jnp.float32)
    for g in range(NG):
        d = (z_sq + p2[:, g * GW:(g + 1) * GW]) + csq_s[0:1, g * GW:(g + 1) * GW]
        lt = d < run_m
        run_m = jnp.minimum(run_m, d)
        run_g = jnp.where(lt, jnp.float32(g), run_g)
    lane = lax.broadcasted_iota(jnp.int32, (BM, GW), 1).astype(jnp.float32)
    kcand = run_g * jnp.float32(GW) + lane
    gm = jnp.min(run_m, axis=1, keepdims=True)
    idxf = jnp.min(jnp.where(run_m == gm, kcand, jnp.float32(KCB)),
                   axis=1, keepdims=True)
    idx_ref[...] = idxf.astype(jnp.int32)


def _distance_argmin_call(z_e, codebook):
    nb = BTOK // BM
    return pl.pallas_call(
        _distance_argmin_body,
        grid=(nb,),
        in_specs=[
            pl.BlockSpec((KCB, DDIM), lambda i: (0, 0)),
            pl.BlockSpec((BM, DDIM), lambda i: (i, 0)),
        ],
        out_specs=pl.BlockSpec((BM, 1), lambda i: (i, 0)),
        out_shape=jax.ShapeDtypeStruct((BTOK, 1), jnp.int32),
        scratch_shapes=[
            pltpu.VMEM((KCB, DDIM), jnp.bfloat16),
            pltpu.VMEM((8, KCB), jnp.float32),
        ],
        compiler_params=pltpu.CompilerParams(
            dimension_semantics=("arbitrary",)),
    )(codebook, z_e)


@functools.lru_cache(maxsize=1)
def _sc_gather_loss_kernel():
    @functools.partial(
        pl.kernel,
        out_type=(
            jax.ShapeDtypeStruct((BTOK, DDIM), jnp.float32),
            jax.ShapeDtypeStruct((NW, LANES), jnp.float32),
        ),
        mesh=plsc.VectorSubcoreMesh(
            core_axis_name="c", subcore_axis_name="s",
            num_cores=NC, num_subcores=NS),
        scratch_types=[
            pltpu.VMEM((BPW,), jnp.int32),
            pltpu.VMEM((SUB, DDIM), jnp.float32),
            pltpu.VMEM((SUB, DDIM), jnp.float32),
            pltpu.VMEM((SUB, DDIM), jnp.float32),
            pltpu.VMEM((SUB, DDIM), jnp.float32),
            pltpu.VMEM((SUB, DDIM), jnp.float32),
            pltpu.VMEM((SUB, DDIM), jnp.float32),
            pltpu.VMEM((LANES,), jnp.float32),
            pltpu.SemaphoreType.DMA,
            pltpu.SemaphoreType.DMA,
            pltpu.SemaphoreType.DMA,
            pltpu.SemaphoreType.DMA,
            pltpu.SemaphoreType.DMA,
            pltpu.SemaphoreType.DMA,
        ],
    )
    def _sc_gather_loss(cb_hbm, idx_hbm, ze_hbm, zq_hbm, part_hbm,
                        idx_v, rows0, rows1, ze0, ze1, st0, st1, acc_v,
                        g0, g1, zs0, zs1, os0, os1):
        wid = lax.axis_index("s") * NC + lax.axis_index("c")
        base = wid * BPW
        pltpu.sync_copy(idx_hbm.at[pl.ds(base, BPW)], idx_v)
        rows = (rows0, rows1)
        zeb = (ze0, ze1)
        stb = (st0, st1)
        gsem = (g0, g1)
        zsem = (zs0, zs1)
        osem = (os0, os1)
        gh = [None] * NSUB
        zh = [None] * NSUB
        oh = [None] * NSUB

        def issue(s):
            b = s % 2
            gh[s] = pltpu.async_copy(
                cb_hbm.at[idx_v.at[pl.ds(s * SUB, SUB)]], rows[b], gsem[b])
            zh[s] = pltpu.async_copy(
                ze_hbm.at[pl.ds(base + s * SUB, SUB)], zeb[b], zsem[b])

        issue(0)
        accs = (jnp.zeros((LANES,), jnp.float32),
                jnp.zeros((LANES,), jnp.float32))
        for s in range(NSUB):
            b = s % 2
            gh[s].wait()
            zh[s].wait()
            if s + 1 < NSUB:
                issue(s + 1)
            if s >= 2:
                oh[s - 2].wait()

            @plsc.parallel_loop(0, SUB, unroll=2, carry=accs)
            def accs(r, a):
                a0, a1 = a
                for cc in range(DDIM // LANES):
                    zq = rows[b][r, pl.ds(cc * LANES, LANES)]
                    ze = zeb[b][r, pl.ds(cc * LANES, LANES)]
                    d = zq - ze
                    stb[b][r, pl.ds(cc * LANES, LANES)] = ze + d
                    if cc % 2 == 0:
                        a0 = a0 + d * d
                    else:
                        a1 = a1 + d * d
                return (a0, a1)

            oh[s] = pltpu.async_copy(
                stb[b], zq_hbm.at[pl.ds(base + s * SUB, SUB)], osem[b])
        oh[NSUB - 2].wait()
        oh[NSUB - 1].wait()
        acc_v[...] = accs[0] + accs[1]
        pltpu.sync_copy(acc_v, part_hbm.at[wid])

    return _sc_gather_loss


def kernel(z_e, codebook):
    idx2 = _distance_argmin_call(z_e, codebook)
    indices = idx2.reshape(BTOK)
    z_q_st, parts = _sc_gather_loss_kernel()(codebook, indices, z_e)
    m = jnp.sum(parts) / (BTOK * DDIM)
    vq_loss = m + BETA_W * m
    return (z_q_st, indices, vq_loss)
